# Initial kernel scaffold; baseline (speedup 1.0000x reference)
import jax
import jax.numpy as jnp
from jax.experimental import pallas as pl


def kernel(G, emb, etypes, V1, comp1, Wself1, b1, V2, comp2, Wself2, b2):
    raise NotImplementedError("write your pallas kernel here")


# speedup vs baseline: 15.1591x; 15.1591x over previous
import functools

import jax
import jax.numpy as jnp
from jax import lax
from jax.experimental import pallas as pl
from jax.experimental.pallas import tpu as pltpu
from jax.experimental.pallas import tpu_sc as plsc

N = 10000
E = 320000
D = 128
R = 8
B = 4

NPAD = 10240
NC = 2
NS = 16
NW = NC * NS
C = 128
K = -(-E // (NW * C))
EPW = K * C
EPAD = NW * EPW

BN = 2048
NB = NPAD // BN


def _table_body(comp_ref, x_ref, v_ref, out_ref):
    r = pl.program_id(1)
    w = (comp_ref[r, 0] * v_ref[0]
         + comp_ref[r, 1] * v_ref[1]
         + comp_ref[r, 2] * v_ref[2]
         + comp_ref[r, 3] * v_ref[3])
    out_ref[...] = jnp.dot(x_ref[...], w, preferred_element_type=jnp.float32)


def _make_table(x, v, comp):
    return pl.pallas_call(
        _table_body,
        grid=(NB, R),
        in_specs=[
            pl.BlockSpec(memory_space=pltpu.SMEM),
            pl.BlockSpec((BN, D), lambda i, r: (i, 0)),
            pl.BlockSpec((B, D, D), lambda i, r: (0, 0, 0)),
        ],
        out_specs=pl.BlockSpec((BN, D), lambda i, r: (r * NB + i, 0)),
        out_shape=jax.ShapeDtypeStruct((R * NPAD, D), jnp.float32),
    )(comp, x, v)


@functools.cache
def _sc_edges_fn():
    mesh = plsc.VectorSubcoreMesh(
        core_axis_name="c", subcore_axis_name="s",
        num_cores=NC, num_subcores=NS)

    @functools.partial(
        pl.kernel,
        out_type=jax.ShapeDtypeStruct((NC * NPAD, D), jnp.float32),
        mesh=mesh,
        scratch_types=[
            pltpu.VMEM((K, C), jnp.int32),
            pltpu.VMEM((K, C), jnp.int32),
            pltpu.VMEM((C, D), jnp.float32),
            pltpu.VMEM_SHARED((NPAD, D), jnp.float32),
            pltpu.SemaphoreType.DMA,
        ],
    )
    def _sc_edges(table_hbm, gidx_hbm, didx_hbm, zeros_hbm, out_hbm,
                  gidx_v, didx_v, rows_v, acc, sem):
        cid = lax.axis_index("c")
        sid = lax.axis_index("s")
        wid = cid * NS + sid
        stripe = NPAD // NS

        pltpu.sync_copy(zeros_hbm.at[pl.ds(sid * stripe, stripe)],
                        acc.at[pl.ds(sid * stripe, stripe)])
        pltpu.sync_copy(gidx_hbm.at[wid], gidx_v)
        pltpu.sync_copy(didx_hbm.at[wid], didx_v)
        plsc.subcore_barrier()

        def chunk(g, carry):
            pltpu.async_copy(table_hbm.at[gidx_v.at[g]], rows_v, sem).wait()
            pltpu.sync_copy(rows_v, acc.at[didx_v.at[g]], add=True)
            return carry

        lax.fori_loop(0, K, chunk, 0)
        plsc.subcore_barrier()

        pltpu.sync_copy(acc.at[pl.ds(sid * stripe, stripe)],
                        out_hbm.at[pl.ds(cid * NPAD + sid * stripe, stripe)])

    return _sc_edges


def _combine_body(p0_ref, p1_ref, x_ref, w_ref, b_ref, out_ref, *, relu):
    h = (p0_ref[...] + p1_ref[...] + b_ref[...]
         + jnp.dot(x_ref[...], w_ref[...], preferred_element_type=jnp.float32))
    out_ref[...] = jnp.maximum(h, 0.0) if relu else h


def _combine(p, x, wself, b, relu):
    return pl.pallas_call(
        functools.partial(_combine_body, relu=relu),
        grid=(NB,),
        in_specs=[
            pl.BlockSpec((BN, D), lambda i: (i, 0)),
            pl.BlockSpec((BN, D), lambda i: (NB + i, 0)),
            pl.BlockSpec((BN, D), lambda i: (i, 0)),
            pl.BlockSpec((D, D), lambda i: (0, 0)),
            pl.BlockSpec((1, D), lambda i: (0, 0)),
        ],
        out_specs=pl.BlockSpec((BN, D), lambda i: (i, 0)),
        out_shape=jax.ShapeDtypeStruct((NPAD, D), jnp.float32),
    )(p, p, x, wself, b.reshape(1, D))


def _layer(x, gidx, didx, zeros, v, comp, wself, b, relu):
    table = _make_table(x, v, comp)
    p = _sc_edges_fn()(table, gidx, didx, zeros)
    return _combine(p, x, wself, b, relu)


def kernel(G, emb, etypes, V1, comp1, Wself1, b1, V2, comp2, Wself2, b2):
    src = G[0].astype(jnp.int32)
    dst = G[1].astype(jnp.int32)
    et = etypes.astype(jnp.int32)

    gidx = et * NPAD + src
    npad_e = EPAD - E
    gidx = jnp.concatenate([gidx, jnp.zeros((npad_e,), jnp.int32)])
    didx = jnp.concatenate([dst, jnp.full((npad_e,), N, jnp.int32)])
    gidx = gidx.reshape(NW, K, C)
    didx = didx.reshape(NW, K, C)

    x0 = jnp.pad(emb, ((0, NPAD - N), (0, 0)))
    zeros = jnp.zeros((NPAD, D), jnp.float32)

    h = _layer(x0, gidx, didx, zeros, V1, comp1, Wself1, b1, True)
    h = _layer(h, gidx, didx, zeros, V2, comp2, Wself2, b2, False)
    return h[:N]
